# baseline (speedup 1.0000x reference)
import functools

import jax
import jax.numpy as jnp
from jax import lax
from jax.experimental import pallas as pl
from jax.experimental.pallas import tpu as pltpu
from jax.experimental.pallas import tpu_sc as plsc

NUM_FIELDS = 26
VOCAB = 100000
EMB_DIM = 16
BATCH = 4096

_NC = 2
_NS = 16
_LANES = 16
_NW = _NC * _NS
_TOTAL = BATCH * NUM_FIELDS
_PER_W = _TOTAL // _NW
_IDX_ROWS = _PER_W // 128
_STEPS = _PER_W // _LANES


def _sc_gather(tab_hbm, idx_hbm, out_hbm, idx_v, rows_v, rows128_v, sem):
    wid = lax.axis_index("s") * _NC + lax.axis_index("c")
    base = wid * _PER_W

    pltpu.sync_copy(idx_hbm.at[wid], idx_v)

    f0 = lax.iota(jnp.int32, _LANES)

    def offset_body(i, f):
        j = i // 8
        c = i - j * 8
        sl = idx_v[j, pl.ds(c * _LANES, _LANES)]
        idx_v[j, pl.ds(c * _LANES, _LANES)] = (sl + f * VOCAB) >> 3
        t = f + _LANES
        return jnp.where(t >= NUM_FIELDS, t - NUM_FIELDS, t)

    lax.fori_loop(0, _STEPS, offset_body, f0)

    def dma_body(j, carry):
        pltpu.async_copy(
            tab_hbm.at[idx_v.at[j]],
            rows128_v,
            sem,
        ).wait()
        return carry

    lax.fori_loop(0, _IDX_ROWS, dma_body, 0)

    pltpu.sync_copy(rows_v, out_hbm.at[pl.ds(base, _PER_W)])


@jax.jit
def _impl(indices, tables):
    tab = tables.reshape(NUM_FIELDS * VOCAB * EMB_DIM // 128, 128)
    idx = indices.reshape(_NW, _IDX_ROWS, 128)
    mesh = plsc.VectorSubcoreMesh(core_axis_name="c", subcore_axis_name="s")
    run = pl.kernel(
        _sc_gather,
        out_type=jax.ShapeDtypeStruct((_TOTAL, EMB_DIM), jnp.float32),
        mesh=mesh,
        compiler_params=pltpu.CompilerParams(use_tc_tiling_on_sc=False),
        scratch_types=[
            pltpu.VMEM((_IDX_ROWS, 128), jnp.int32),
            pltpu.VMEM((_PER_W, EMB_DIM), jnp.float32),
            pltpu.VMEM((128, 128), jnp.float32),
            pltpu.SemaphoreType.DMA,
        ],
    )
    out = run(tab, idx)
    return out.reshape(BATCH, NUM_FIELDS * EMB_DIM)


def kernel(indices, tables):
    return _impl(indices, tables)


# speedup vs baseline: 1.8830x; 1.8830x over previous
import functools

import jax
import jax.numpy as jnp
from jax import lax
from jax.experimental import pallas as pl
from jax.experimental.pallas import tpu as pltpu
from jax.experimental.pallas import tpu_sc as plsc

NUM_FIELDS = 26
VOCAB = 100000
EMB_DIM = 16
BATCH = 4096

_NC = 2
_NS = 16
_NW = _NC * _NS
_ROWS = NUM_FIELDS * EMB_DIM
_R_PER_W = _ROWS // _NW
_CHUNKS = BATCH // 128


def _sc_gather(tab_hbm, idx_hbm, out_hbm, idx_v, row_v, sem):
    wid = lax.axis_index("s") * _NC + lax.axis_index("c")

    def row_body(j, carry):
        r = wid * _R_PER_W + j
        f = r // EMB_DIM
        d = r - f * EMB_DIM
        pltpu.sync_copy(idx_hbm.at[f], idx_v)
        src = tab_hbm.at[f].at[d]

        def chunk_body(k, c2):
            pltpu.async_copy(
                src.at[idx_v.at[k]],
                row_v.at[pl.ds(k * 128, 128)],
                sem,
            ).wait()
            return c2

        lax.fori_loop(0, _CHUNKS, chunk_body, 0)
        pltpu.sync_copy(row_v, out_hbm.at[r])
        return carry

    lax.fori_loop(0, _R_PER_W, row_body, 0)


@jax.jit
def _impl(indices, tables):
    tabT = jnp.transpose(tables, (0, 2, 1))
    idxT = indices.T.reshape(NUM_FIELDS, _CHUNKS, 128)
    mesh = plsc.VectorSubcoreMesh(core_axis_name="c", subcore_axis_name="s")
    run = pl.kernel(
        _sc_gather,
        out_type=jax.ShapeDtypeStruct((_ROWS, BATCH), jnp.float32),
        mesh=mesh,
        compiler_params=pltpu.CompilerParams(use_tc_tiling_on_sc=False),
        scratch_types=[
            pltpu.VMEM((_CHUNKS, 128), jnp.int32),
            pltpu.VMEM((BATCH,), jnp.float32),
            pltpu.SemaphoreType.DMA,
        ],
    )
    out = run(tabT, idxT)
    return out.T.reshape(BATCH, NUM_FIELDS * EMB_DIM)


def kernel(indices, tables):
    return _impl(indices, tables)


# speedup vs baseline: 3.1636x; 1.6801x over previous
import functools

import jax
import jax.numpy as jnp
from jax import lax
from jax.experimental import pallas as pl
from jax.experimental.pallas import tpu as pltpu
from jax.experimental.pallas import tpu_sc as plsc

NUM_FIELDS = 26
VOCAB = 100000
EMB_DIM = 16
BATCH = 4096

_NC = 2
_NS = 16
_NW = _NC * _NS
_ROWS = NUM_FIELDS * EMB_DIM
_R_PER_W = _ROWS // _NW
_CHUNKS = BATCH // 128
_NDMA = _R_PER_W * _CHUNKS
_LAG = 16


def _sc_gather(tab_hbm, idx_hbm, out_hbm, idx_v, rows_v, sem):
    wid = lax.axis_index("s") * _NC + lax.axis_index("c")
    r0 = wid * _R_PER_W
    f0 = r0 // EMB_DIM

    pltpu.sync_copy(idx_hbm.at[f0], idx_v.at[0])
    f1 = (r0 + _R_PER_W - 1) // EMB_DIM
    pltpu.sync_copy(idx_hbm.at[f1], idx_v.at[1])

    def slices(i):
        row = i // _CHUNKS
        chunk = i - row * _CHUNKS
        r = r0 + row
        f = r // EMB_DIM
        d = r - f * EMB_DIM
        src = tab_hbm.at[f].at[d].at[idx_v.at[f - f0].at[chunk]]
        dst = rows_v.at[pl.ds(i * 128, 128)]
        return src, dst

    def fire_body(i, carry):
        src, dst = slices(i)
        pltpu.async_copy(src, dst, sem)

        @pl.when(i >= _LAG)
        def _():
            src2, dst2 = slices(i - _LAG)
            pltpu.make_async_copy(src2, dst2, sem).wait()

        return carry

    lax.fori_loop(0, _NDMA, fire_body, 0)

    def drain_body(i, carry):
        src2, dst2 = slices(i)
        pltpu.make_async_copy(src2, dst2, sem).wait()
        return carry

    lax.fori_loop(_NDMA - _LAG, _NDMA, drain_body, 0)

    pltpu.sync_copy(rows_v, out_hbm.at[pl.ds(wid * _NDMA * 128, _NDMA * 128)])


@jax.jit
def _impl(indices, tables):
    tabT = jnp.transpose(tables, (0, 2, 1))
    idxT = indices.T.reshape(NUM_FIELDS, _CHUNKS, 128)
    mesh = plsc.VectorSubcoreMesh(core_axis_name="c", subcore_axis_name="s")
    run = pl.kernel(
        _sc_gather,
        out_type=jax.ShapeDtypeStruct((_ROWS * BATCH,), jnp.float32),
        mesh=mesh,
        compiler_params=pltpu.CompilerParams(use_tc_tiling_on_sc=False),
        scratch_types=[
            pltpu.VMEM((2, _CHUNKS, 128), jnp.int32),
            pltpu.VMEM((_NDMA * 128,), jnp.float32),
            pltpu.SemaphoreType.DMA,
        ],
    )
    out = run(tabT, idxT)
    return out.reshape(_ROWS, BATCH).T.reshape(BATCH, NUM_FIELDS * EMB_DIM)


def kernel(indices, tables):
    return _impl(indices, tables)


# speedup vs baseline: 3.2641x; 1.0318x over previous
import functools

import jax
import jax.numpy as jnp
from jax import lax
from jax.experimental import pallas as pl
from jax.experimental.pallas import tpu as pltpu
from jax.experimental.pallas import tpu_sc as plsc

NUM_FIELDS = 26
VOCAB = 100000
EMB_DIM = 16
BATCH = 4096

_NC = 2
_NS = 16
_NW = _NC * _NS
_ROWS = NUM_FIELDS * EMB_DIM
_R_PER_W = _ROWS // _NW
_CHUNKS = BATCH // 128
_NDMA = _R_PER_W * _CHUNKS
_LAG = 48


def _sc_gather(tab_hbm, idx_hbm, out_hbm, idx_v, rows_v, sem):
    wid = lax.axis_index("s") * _NC + lax.axis_index("c")
    r0 = wid * _R_PER_W
    f0 = r0 // EMB_DIM

    pltpu.sync_copy(idx_hbm.at[f0], idx_v.at[0])
    f1 = (r0 + _R_PER_W - 1) // EMB_DIM
    pltpu.sync_copy(idx_hbm.at[f1], idx_v.at[1])

    def slices(i):
        row = i // _CHUNKS
        chunk = i - row * _CHUNKS
        r = r0 + row
        f = r // EMB_DIM
        d = r - f * EMB_DIM
        src = tab_hbm.at[f].at[d].at[idx_v.at[f - f0].at[chunk]]
        dst = rows_v.at[pl.ds(i * 128, 128)]
        return src, dst

    def fire_body(i, carry):
        src, dst = slices(i)
        pltpu.async_copy(src, dst, sem)

        @pl.when(i >= _LAG)
        def _():
            src2, dst2 = slices(i - _LAG)
            pltpu.make_async_copy(src2, dst2, sem).wait()

        return carry

    lax.fori_loop(0, _NDMA, fire_body, 0)

    def drain_body(i, carry):
        src2, dst2 = slices(i)
        pltpu.make_async_copy(src2, dst2, sem).wait()
        return carry

    lax.fori_loop(_NDMA - _LAG, _NDMA, drain_body, 0)

    pltpu.sync_copy(rows_v, out_hbm.at[pl.ds(wid * _NDMA * 128, _NDMA * 128)])


@jax.jit
def _impl(indices, tables):
    tabT = jnp.transpose(tables, (0, 2, 1))
    idxT = indices.T.reshape(NUM_FIELDS, _CHUNKS, 128)
    mesh = plsc.VectorSubcoreMesh(core_axis_name="c", subcore_axis_name="s")
    run = pl.kernel(
        _sc_gather,
        out_type=jax.ShapeDtypeStruct((_ROWS * BATCH,), jnp.float32),
        mesh=mesh,
        compiler_params=pltpu.CompilerParams(use_tc_tiling_on_sc=False),
        scratch_types=[
            pltpu.VMEM((2, _CHUNKS, 128), jnp.int32),
            pltpu.VMEM((_NDMA * 128,), jnp.float32),
            pltpu.SemaphoreType.DMA,
        ],
    )
    out = run(tabT, idxT)
    return out.reshape(_ROWS, BATCH).T.reshape(BATCH, NUM_FIELDS * EMB_DIM)


def kernel(indices, tables):
    return _impl(indices, tables)
